# baseline (speedup 1.0000x reference)
import functools

import jax
import jax.numpy as jnp
from jax.experimental import pallas as pl
from jax.experimental.pallas import tpu as pltpu

F32 = jnp.float32
HID = 96


def _sigmoid(x):
    return 1.0 / (1.0 + jnp.exp(-x))


def _silu(x):
    return x * _sigmoid(x)


def _mm_body(x_ref, w_ref, o_ref):
    o_ref[...] = jax.lax.dot_general(
        x_ref[...], w_ref[...], (((1,), (0,)), ((), ())),
        preferred_element_type=F32)


def _matmul(x, w, block):
    n, k = x.shape
    kw, wout = w.shape
    return pl.pallas_call(
        _mm_body,
        grid=(n // block,),
        in_specs=[pl.BlockSpec((block, k), lambda i: (i, 0)),
                  pl.BlockSpec((kw, wout), lambda i: (0, 0))],
        out_specs=pl.BlockSpec((block, wout), lambda i: (i, 0)),
        out_shape=jax.ShapeDtypeStruct((n, wout), F32),
    )(x, w)


def _edge_body(ga_ref, gb_ref, gv_ref, e_ref, c_ref, sig_ref, sv_ref):
    eh = ga_ref[...] + gb_ref[...] + jax.lax.dot_general(
        e_ref[...], c_ref[...], (((1,), (0,)), ((), ())),
        preferred_element_type=F32)
    sig = _sigmoid(eh)
    sig_ref[...] = sig
    sv_ref[...] = sig * gv_ref[...]


def _edge_stage(ga, gb, gv, e, c, block):
    n = ga.shape[0]
    bs = pl.BlockSpec((block, HID), lambda i: (i, 0))
    return pl.pallas_call(
        _edge_body,
        grid=(n // block,),
        in_specs=[bs, bs, bs, bs, pl.BlockSpec((HID, HID), lambda i: (0, 0))],
        out_specs=[bs, bs],
        out_shape=[jax.ShapeDtypeStruct((n, HID), F32),
                   jax.ShapeDtypeStruct((n, HID), F32)],
    )(ga, gb, gv, e, c)


def _ns1_body(u_ref, num_ref, den_ref, s_ref, ps_ref, pq_ref):
    s = u_ref[...] + num_ref[...] / (den_ref[...] + 1e-6)
    s_ref[...] = s
    ps_ref[...] = jnp.sum(s, axis=0, keepdims=True)[None]
    pq_ref[...] = jnp.sum(s * s, axis=0, keepdims=True)[None]


def _node_stage1(hu, num, den, block):
    n = hu.shape[0]
    nb = n // block
    bs = pl.BlockSpec((block, HID), lambda i: (i, 0))
    rs = pl.BlockSpec((1, 1, HID), lambda i: (i, 0, 0))
    return pl.pallas_call(
        _ns1_body,
        grid=(nb,),
        in_specs=[bs, bs, bs],
        out_specs=[bs, rs, rs],
        out_shape=[jax.ShapeDtypeStruct((n, HID), F32),
                   jax.ShapeDtypeStruct((nb, 1, HID), F32),
                   jax.ShapeDtypeStruct((nb, 1, HID), F32)],
    )(hu, num, den)


def _ns2_body(h_ref, s_ref, mu_ref, iv_ref, g_ref, b_ref, o_ref):
    xn = (s_ref[...] - mu_ref[...]) * iv_ref[...] * g_ref[...] + b_ref[...]
    o_ref[...] = h_ref[...] + _silu(xn)


def _node_stage2(h, s, mu, iv, g, b, block):
    n = h.shape[0]
    bs = pl.BlockSpec((block, HID), lambda i: (i, 0))
    rs = pl.BlockSpec((1, HID), lambda i: (0, 0))
    return pl.pallas_call(
        _ns2_body,
        grid=(n // block,),
        in_specs=[bs, bs, rs, rs, rs, rs],
        out_specs=bs,
        out_shape=jax.ShapeDtypeStruct((n, HID), F32),
    )(h, s, mu, iv, g, b)


def _finish_bn(ps, pq, n):
    mu = jnp.sum(ps, axis=(0, 1), keepdims=False)[None, :] / n
    var = jnp.sum(pq, axis=(0, 1), keepdims=False)[None, :] / n - mu * mu
    iv = jax.lax.rsqrt(var + 1e-5)
    return mu, iv


def _emb_body(x_ref, w_ref, b_ref, s_ref, ps_ref, pq_ref):
    s = jax.lax.dot_general(
        x_ref[...], w_ref[...], (((1,), (0,)), ((), ())),
        preferred_element_type=F32) + b_ref[...]
    s_ref[...] = s
    ps_ref[...] = jnp.sum(s, axis=0, keepdims=True)[None]
    pq_ref[...] = jnp.sum(s * s, axis=0, keepdims=True)[None]


def _emb_stage1(x, w, b, block):
    n, k = x.shape
    nb = n // block
    rs = pl.BlockSpec((1, 1, HID), lambda i: (i, 0, 0))
    return pl.pallas_call(
        _emb_body,
        grid=(nb,),
        in_specs=[pl.BlockSpec((block, k), lambda i: (i, 0)),
                  pl.BlockSpec((k, HID), lambda i: (0, 0)),
                  pl.BlockSpec((1, HID), lambda i: (0, 0))],
        out_specs=[pl.BlockSpec((block, HID), lambda i: (i, 0)), rs, rs],
        out_shape=[jax.ShapeDtypeStruct((n, HID), F32),
                   jax.ShapeDtypeStruct((nb, 1, HID), F32),
                   jax.ShapeDtypeStruct((nb, 1, HID), F32)],
    )(x, w, b)


def _bnact_body(s_ref, mu_ref, iv_ref, g_ref, b_ref, o_ref):
    xn = (s_ref[...] - mu_ref[...]) * iv_ref[...] * g_ref[...] + b_ref[...]
    o_ref[...] = _silu(xn)


def _bnact(s, mu, iv, g, b, block):
    n = s.shape[0]
    bs = pl.BlockSpec((block, HID), lambda i: (i, 0))
    rs = pl.BlockSpec((1, HID), lambda i: (0, 0))
    return pl.pallas_call(
        _bnact_body,
        grid=(n // block,),
        in_specs=[bs, rs, rs, rs, rs],
        out_specs=bs,
        out_shape=jax.ShapeDtypeStruct((n, HID), F32),
    )(s, mu, iv, g, b)


def _smear_body(d_ref, o_ref, *, start, delta):
    off = start + delta * jax.lax.broadcasted_iota(
        jnp.int32, (1, HID), 1).astype(F32)
    diff = d_ref[...] - off
    o_ref[...] = jnp.exp((-0.5 / (delta * delta)) * diff * diff)


def _smearing(d, start, stop, block):
    n = d.shape[0]
    delta = (stop - start) / (HID - 1)
    return pl.pallas_call(
        functools.partial(_smear_body, start=start, delta=delta),
        grid=(n // block,),
        in_specs=[pl.BlockSpec((block, 1), lambda i: (i, 0))],
        out_specs=pl.BlockSpec((block, HID), lambda i: (i, 0)),
        out_shape=jax.ShapeDtypeStruct((n, HID), F32),
    )(d.reshape(n, 1))


def _heads_body(c_ref, w1_ref, b1_ref, w2_ref, b2_ref,
                w3_ref, b3_ref, w4_ref, b4_ref, o1_ref, o2_ref):
    c = c_ref[...]
    t1 = jax.lax.dot_general(c, w1_ref[...], (((1,), (0,)), ((), ())),
                             preferred_element_type=F32) + b1_ref[...]
    o1_ref[...] = jax.lax.dot_general(_silu(t1), w2_ref[...],
                                      (((1,), (0,)), ((), ())),
                                      preferred_element_type=F32) + b2_ref[...]
    t2 = jax.lax.dot_general(c, w3_ref[...], (((1,), (0,)), ((), ())),
                             preferred_element_type=F32) + b3_ref[...]
    o2_ref[...] = jax.lax.dot_general(_silu(t2), w4_ref[...],
                                      (((1,), (0,)), ((), ())),
                                      preferred_element_type=F32) + b4_ref[...]


def _heads(c, w1, b1, w2, b2, w3, b3, w4, b4):
    g = c.shape[0]
    full = lambda shape: pl.BlockSpec(shape, lambda: tuple(0 for _ in shape))
    return pl.pallas_call(
        _heads_body,
        in_specs=[full((g, HID)),
                  full((HID, HID)), full((1, HID)), full((HID, 1)), full((1, 1)),
                  full((HID, HID)), full((1, HID)), full((HID, 1)), full((1, 1))],
        out_specs=[full((g, 1)), full((g, 1))],
        out_shape=[jax.ShapeDtypeStruct((g, 1), F32),
                   jax.ShapeDtypeStruct((g, 1), F32)],
    )(c, w1, b1.reshape(1, HID), w2, b2.reshape(1, 1),
      w3, b3.reshape(1, HID), w4, b4.reshape(1, 1))


def _gated_layer(h, e, src, dst, p, eblock, nblock):
    n = h.shape[0]
    wcat = jnp.concatenate([p['A'], p['B'], p['V'], p['U']], axis=1)
    hw = _matmul(h, wcat, nblock)
    ga = jnp.take(hw[:, 0 * HID:1 * HID], src, axis=0)
    gb = jnp.take(hw[:, 1 * HID:2 * HID], dst, axis=0)
    gv = jnp.take(hw[:, 2 * HID:3 * HID], src, axis=0)
    sigma, sv = _edge_stage(ga, gb, gv, e, p['C'], eblock)
    num = jax.ops.segment_sum(sv, dst, num_segments=n)
    den = jax.ops.segment_sum(sigma, dst, num_segments=n)
    s, ps, pq = _node_stage1(hw[:, 3 * HID:4 * HID], num, den, nblock)
    mu, iv = _finish_bn(ps, pq, n)
    return _node_stage2(h, s, mu, iv,
                        p['bn_g'].reshape(1, HID), p['bn_b'].reshape(1, HID),
                        nblock)


def kernel(x, edge_attr, angle_attr, edge_index, edge_index_lg, batch, params):
    n_nodes = x.shape[0]
    n_edges = edge_attr.shape[0]
    n_graphs = 256

    nblock = 2000
    eblock = 2000

    t, ps, pq = _emb_stage1(x, params['emb_w'],
                            params['emb_b'].reshape(1, HID), nblock)
    mu, iv = _finish_bn(ps, pq, n_nodes)
    h = _bnact(t, mu, iv, params['emb_bn_g'].reshape(1, HID),
               params['emb_bn_b'].reshape(1, HID), nblock)

    m = _smearing(edge_attr, 0.0, 6.0, eblock)
    a = _smearing(angle_attr, -1.0, 180.0, eblock)

    src, dst = edge_index[0], edge_index[1]
    src_lg, dst_lg = edge_index_lg[0], edge_index_lg[1]

    for i in range(len(params['atom'])):
        m = _gated_layer(m, a, src_lg, dst_lg, params['line'][i],
                         eblock, eblock)
        h = _gated_layer(h, m, src, dst, params['atom'][i],
                         eblock, nblock)

    ones = jnp.ones((n_nodes,), F32)
    counts = jax.ops.segment_sum(ones, batch, num_segments=n_graphs)
    c = (jax.ops.segment_sum(h, batch, num_segments=n_graphs)
         / jnp.maximum(counts, 1.0)[:, None])

    out_bg, out_hull = _heads(
        c, params['bg_w1'], params['bg_b1'], params['bg_w2'], params['bg_b2'],
        params['hull_w1'], params['hull_b1'], params['hull_w2'],
        params['hull_b2'])
    return (out_bg, out_hull)


# speedup vs baseline: 1.0496x; 1.0496x over previous
import functools

import jax
import jax.numpy as jnp
from jax.experimental import pallas as pl
from jax.experimental.pallas import tpu as pltpu

F32 = jnp.float32
HID = 96


def _sigmoid(x):
    return 1.0 / (1.0 + jnp.exp(-x))


def _silu(x):
    return x * _sigmoid(x)


BF16 = jnp.bfloat16


def _mm_body(x_ref, w_ref, o_ref):
    o_ref[...] = jax.lax.dot_general(
        x_ref[...], w_ref[...], (((1,), (0,)), ((), ())),
        preferred_element_type=F32).astype(o_ref.dtype)


def _matmul(x, w, block, out_dtype=F32):
    n, k = x.shape
    kw, wout = w.shape
    return pl.pallas_call(
        _mm_body,
        grid=(n // block,),
        in_specs=[pl.BlockSpec((block, k), lambda i: (i, 0)),
                  pl.BlockSpec((kw, wout), lambda i: (0, 0))],
        out_specs=pl.BlockSpec((block, wout), lambda i: (i, 0)),
        out_shape=jax.ShapeDtypeStruct((n, wout), out_dtype),
    )(x, w)


def _edge_body(ga_ref, gb_ref, gv_ref, e_ref, c_ref, sig_ref, sv_ref):
    eh = (ga_ref[...].astype(F32) + gb_ref[...].astype(F32)
          + jax.lax.dot_general(
              e_ref[...], c_ref[...], (((1,), (0,)), ((), ())),
              preferred_element_type=F32))
    sig = _sigmoid(eh)
    sig_ref[...] = sig.astype(BF16)
    sv_ref[...] = (sig * gv_ref[...].astype(F32)).astype(BF16)


def _edge_stage(ga, gb, gv, e, c, block):
    n = ga.shape[0]
    bs = pl.BlockSpec((block, HID), lambda i: (i, 0))
    return pl.pallas_call(
        _edge_body,
        grid=(n // block,),
        in_specs=[bs, bs, bs, bs, pl.BlockSpec((HID, HID), lambda i: (0, 0))],
        out_specs=[bs, bs],
        out_shape=[jax.ShapeDtypeStruct((n, HID), BF16),
                   jax.ShapeDtypeStruct((n, HID), BF16)],
    )(ga, gb, gv, e, c)


def _ns1_body(u_ref, num_ref, den_ref, s_ref, ps_ref, pq_ref):
    s = (u_ref[...].astype(F32)
         + num_ref[...].astype(F32) / (den_ref[...].astype(F32) + 1e-6))
    s_ref[...] = s
    ps_ref[...] = jnp.sum(s, axis=0, keepdims=True)[None]
    pq_ref[...] = jnp.sum(s * s, axis=0, keepdims=True)[None]


def _node_stage1(hu, num, den, block):
    n = hu.shape[0]
    nb = n // block
    bs = pl.BlockSpec((block, HID), lambda i: (i, 0))
    rs = pl.BlockSpec((1, 1, HID), lambda i: (i, 0, 0))
    return pl.pallas_call(
        _ns1_body,
        grid=(nb,),
        in_specs=[bs, bs, bs],
        out_specs=[bs, rs, rs],
        out_shape=[jax.ShapeDtypeStruct((n, HID), F32),
                   jax.ShapeDtypeStruct((nb, 1, HID), F32),
                   jax.ShapeDtypeStruct((nb, 1, HID), F32)],
    )(hu, num, den)


def _ns2_body(h_ref, s_ref, mu_ref, iv_ref, g_ref, b_ref, o_ref):
    xn = (s_ref[...] - mu_ref[...]) * iv_ref[...] * g_ref[...] + b_ref[...]
    o_ref[...] = h_ref[...] + _silu(xn)


def _node_stage2(h, s, mu, iv, g, b, block):
    n = h.shape[0]
    bs = pl.BlockSpec((block, HID), lambda i: (i, 0))
    rs = pl.BlockSpec((1, HID), lambda i: (0, 0))
    return pl.pallas_call(
        _ns2_body,
        grid=(n // block,),
        in_specs=[bs, bs, rs, rs, rs, rs],
        out_specs=bs,
        out_shape=jax.ShapeDtypeStruct((n, HID), F32),
    )(h, s, mu, iv, g, b)


def _finish_bn(ps, pq, n):
    mu = jnp.sum(ps, axis=(0, 1), keepdims=False)[None, :] / n
    var = jnp.sum(pq, axis=(0, 1), keepdims=False)[None, :] / n - mu * mu
    iv = jax.lax.rsqrt(var + 1e-5)
    return mu, iv


def _emb_body(x_ref, w_ref, b_ref, s_ref, ps_ref, pq_ref):
    s = jax.lax.dot_general(
        x_ref[...], w_ref[...], (((1,), (0,)), ((), ())),
        preferred_element_type=F32) + b_ref[...]
    s_ref[...] = s
    ps_ref[...] = jnp.sum(s, axis=0, keepdims=True)[None]
    pq_ref[...] = jnp.sum(s * s, axis=0, keepdims=True)[None]


def _emb_stage1(x, w, b, block):
    n, k = x.shape
    nb = n // block
    rs = pl.BlockSpec((1, 1, HID), lambda i: (i, 0, 0))
    return pl.pallas_call(
        _emb_body,
        grid=(nb,),
        in_specs=[pl.BlockSpec((block, k), lambda i: (i, 0)),
                  pl.BlockSpec((k, HID), lambda i: (0, 0)),
                  pl.BlockSpec((1, HID), lambda i: (0, 0))],
        out_specs=[pl.BlockSpec((block, HID), lambda i: (i, 0)), rs, rs],
        out_shape=[jax.ShapeDtypeStruct((n, HID), F32),
                   jax.ShapeDtypeStruct((nb, 1, HID), F32),
                   jax.ShapeDtypeStruct((nb, 1, HID), F32)],
    )(x, w, b)


def _bnact_body(s_ref, mu_ref, iv_ref, g_ref, b_ref, o_ref):
    xn = (s_ref[...] - mu_ref[...]) * iv_ref[...] * g_ref[...] + b_ref[...]
    o_ref[...] = _silu(xn)


def _bnact(s, mu, iv, g, b, block):
    n = s.shape[0]
    bs = pl.BlockSpec((block, HID), lambda i: (i, 0))
    rs = pl.BlockSpec((1, HID), lambda i: (0, 0))
    return pl.pallas_call(
        _bnact_body,
        grid=(n // block,),
        in_specs=[bs, rs, rs, rs, rs],
        out_specs=bs,
        out_shape=jax.ShapeDtypeStruct((n, HID), F32),
    )(s, mu, iv, g, b)


def _smear_body(d_ref, o_ref, *, start, delta):
    off = start + delta * jax.lax.broadcasted_iota(
        jnp.int32, (1, HID), 1).astype(F32)
    diff = d_ref[...] - off
    o_ref[...] = jnp.exp((-0.5 / (delta * delta)) * diff * diff)


def _smearing(d, start, stop, block):
    n = d.shape[0]
    delta = (stop - start) / (HID - 1)
    return pl.pallas_call(
        functools.partial(_smear_body, start=start, delta=delta),
        grid=(n // block,),
        in_specs=[pl.BlockSpec((block, 1), lambda i: (i, 0))],
        out_specs=pl.BlockSpec((block, HID), lambda i: (i, 0)),
        out_shape=jax.ShapeDtypeStruct((n, HID), F32),
    )(d.reshape(n, 1))


def _heads_body(c_ref, w1_ref, b1_ref, w2_ref, b2_ref,
                w3_ref, b3_ref, w4_ref, b4_ref, o1_ref, o2_ref):
    c = c_ref[...]
    t1 = jax.lax.dot_general(c, w1_ref[...], (((1,), (0,)), ((), ())),
                             preferred_element_type=F32) + b1_ref[...]
    o1_ref[...] = jax.lax.dot_general(_silu(t1), w2_ref[...],
                                      (((1,), (0,)), ((), ())),
                                      preferred_element_type=F32) + b2_ref[...]
    t2 = jax.lax.dot_general(c, w3_ref[...], (((1,), (0,)), ((), ())),
                             preferred_element_type=F32) + b3_ref[...]
    o2_ref[...] = jax.lax.dot_general(_silu(t2), w4_ref[...],
                                      (((1,), (0,)), ((), ())),
                                      preferred_element_type=F32) + b4_ref[...]


def _heads(c, w1, b1, w2, b2, w3, b3, w4, b4):
    g = c.shape[0]
    full = lambda shape: pl.BlockSpec(shape, lambda: tuple(0 for _ in shape))
    return pl.pallas_call(
        _heads_body,
        in_specs=[full((g, HID)),
                  full((HID, HID)), full((1, HID)), full((HID, 1)), full((1, 1)),
                  full((HID, HID)), full((1, HID)), full((HID, 1)), full((1, 1))],
        out_specs=[full((g, 1)), full((g, 1))],
        out_shape=[jax.ShapeDtypeStruct((g, 1), F32),
                   jax.ShapeDtypeStruct((g, 1), F32)],
    )(c, w1, b1.reshape(1, HID), w2, b2.reshape(1, 1),
      w3, b3.reshape(1, HID), w4, b4.reshape(1, 1))


def _gated_layer(h, e, src, dst, p, eblock, nblock):
    n = h.shape[0]
    wabv = jnp.concatenate([p['A'], p['B'], p['V']], axis=1)
    hw = _matmul(h, wabv, nblock, out_dtype=BF16)
    hu = _matmul(h, p['U'], nblock)
    ga = jnp.take(hw[:, 0 * HID:1 * HID], src, axis=0)
    gb = jnp.take(hw[:, 1 * HID:2 * HID], dst, axis=0)
    gv = jnp.take(hw[:, 2 * HID:3 * HID], src, axis=0)
    sigma, sv = _edge_stage(ga, gb, gv, e, p['C'], eblock)
    num = jax.ops.segment_sum(sv, dst, num_segments=n)
    den = jax.ops.segment_sum(sigma, dst, num_segments=n)
    s, ps, pq = _node_stage1(hu, num, den, nblock)
    mu, iv = _finish_bn(ps, pq, n)
    return _node_stage2(h, s, mu, iv,
                        p['bn_g'].reshape(1, HID), p['bn_b'].reshape(1, HID),
                        nblock)


def kernel(x, edge_attr, angle_attr, edge_index, edge_index_lg, batch, params):
    n_nodes = x.shape[0]
    n_edges = edge_attr.shape[0]
    n_graphs = 256

    nblock = 2000
    eblock = 2000

    t, ps, pq = _emb_stage1(x, params['emb_w'],
                            params['emb_b'].reshape(1, HID), nblock)
    mu, iv = _finish_bn(ps, pq, n_nodes)
    h = _bnact(t, mu, iv, params['emb_bn_g'].reshape(1, HID),
               params['emb_bn_b'].reshape(1, HID), nblock)

    m = _smearing(edge_attr, 0.0, 6.0, eblock)
    a = _smearing(angle_attr, -1.0, 180.0, eblock)

    src, dst = edge_index[0], edge_index[1]
    src_lg, dst_lg = edge_index_lg[0], edge_index_lg[1]

    for i in range(len(params['atom'])):
        m = _gated_layer(m, a, src_lg, dst_lg, params['line'][i],
                         eblock, eblock)
        h = _gated_layer(h, m, src, dst, params['atom'][i],
                         eblock, nblock)

    ones = jnp.ones((n_nodes,), F32)
    counts = jax.ops.segment_sum(ones, batch, num_segments=n_graphs)
    c = (jax.ops.segment_sum(h, batch, num_segments=n_graphs)
         / jnp.maximum(counts, 1.0)[:, None])

    out_bg, out_hull = _heads(
        c, params['bg_w1'], params['bg_b1'], params['bg_w2'], params['bg_b2'],
        params['hull_w1'], params['hull_b1'], params['hull_w2'],
        params['hull_b2'])
    return (out_bg, out_hull)


# speedup vs baseline: 1.1068x; 1.0545x over previous
import functools

import jax
import jax.numpy as jnp
from jax.experimental import pallas as pl
from jax.experimental.pallas import tpu as pltpu

F32 = jnp.float32
HID = 96


def _sigmoid(x):
    return 1.0 / (1.0 + jnp.exp(-x))


def _silu(x):
    return x * _sigmoid(x)


BF16 = jnp.bfloat16


def _mm_body(x_ref, w_ref, o_ref):
    o_ref[...] = jax.lax.dot_general(
        x_ref[...], w_ref[...], (((1,), (0,)), ((), ())),
        preferred_element_type=F32).astype(o_ref.dtype)


def _matmul(x, w, block, out_dtype=F32):
    n, k = x.shape
    kw, wout = w.shape
    return pl.pallas_call(
        _mm_body,
        grid=(n // block,),
        in_specs=[pl.BlockSpec((block, k), lambda i: (i, 0)),
                  pl.BlockSpec((kw, wout), lambda i: (0, 0))],
        out_specs=pl.BlockSpec((block, wout), lambda i: (i, 0)),
        out_shape=jax.ShapeDtypeStruct((n, wout), out_dtype),
    )(x, w)


def _edge_body(ga_ref, gb_ref, gv_ref, e_ref, c_ref, sig_ref, sv_ref):
    eh = (ga_ref[...].astype(F32) + gb_ref[...].astype(F32)
          + jax.lax.dot_general(
              e_ref[...], c_ref[...], (((1,), (0,)), ((), ())),
              preferred_element_type=F32))
    sig = _sigmoid(eh)
    sig_ref[...] = sig
    sv_ref[...] = sig * gv_ref[...].astype(F32)


def _edge_stage(ga, gb, gv, e, c, block):
    n = ga.shape[0]
    bs = pl.BlockSpec((block, HID), lambda i: (i, 0))
    return pl.pallas_call(
        _edge_body,
        grid=(n // block,),
        in_specs=[bs, bs, bs, bs, pl.BlockSpec((HID, HID), lambda i: (0, 0))],
        out_specs=[bs, bs],
        out_shape=[jax.ShapeDtypeStruct((n, HID), F32),
                   jax.ShapeDtypeStruct((n, HID), F32)],
    )(ga, gb, gv, e, c)


def _ns1_body(u_ref, num_ref, den_ref, s_ref, ps_ref, pq_ref):
    s = (u_ref[...].astype(F32)
         + num_ref[...].astype(F32) / (den_ref[...].astype(F32) + 1e-6))
    s_ref[...] = s
    ps_ref[...] = jnp.sum(s, axis=0, keepdims=True)[None]
    pq_ref[...] = jnp.sum(s * s, axis=0, keepdims=True)[None]


def _node_stage1(hu, num, den, block):
    n = hu.shape[0]
    nb = n // block
    bs = pl.BlockSpec((block, HID), lambda i: (i, 0))
    rs = pl.BlockSpec((1, 1, HID), lambda i: (i, 0, 0))
    return pl.pallas_call(
        _ns1_body,
        grid=(nb,),
        in_specs=[bs, bs, bs],
        out_specs=[bs, rs, rs],
        out_shape=[jax.ShapeDtypeStruct((n, HID), F32),
                   jax.ShapeDtypeStruct((nb, 1, HID), F32),
                   jax.ShapeDtypeStruct((nb, 1, HID), F32)],
    )(hu, num, den)


def _ns2_body(h_ref, s_ref, mu_ref, iv_ref, g_ref, b_ref, o_ref):
    xn = (s_ref[...] - mu_ref[...]) * iv_ref[...] * g_ref[...] + b_ref[...]
    o_ref[...] = h_ref[...] + _silu(xn)


def _node_stage2(h, s, mu, iv, g, b, block):
    n = h.shape[0]
    bs = pl.BlockSpec((block, HID), lambda i: (i, 0))
    rs = pl.BlockSpec((1, HID), lambda i: (0, 0))
    return pl.pallas_call(
        _ns2_body,
        grid=(n // block,),
        in_specs=[bs, bs, rs, rs, rs, rs],
        out_specs=bs,
        out_shape=jax.ShapeDtypeStruct((n, HID), F32),
    )(h, s, mu, iv, g, b)


def _finish_bn(ps, pq, n):
    mu = jnp.sum(ps, axis=(0, 1), keepdims=False)[None, :] / n
    var = jnp.sum(pq, axis=(0, 1), keepdims=False)[None, :] / n - mu * mu
    iv = jax.lax.rsqrt(var + 1e-5)
    return mu, iv


def _emb_body(x_ref, w_ref, b_ref, s_ref, ps_ref, pq_ref):
    s = jax.lax.dot_general(
        x_ref[...], w_ref[...], (((1,), (0,)), ((), ())),
        preferred_element_type=F32) + b_ref[...]
    s_ref[...] = s
    ps_ref[...] = jnp.sum(s, axis=0, keepdims=True)[None]
    pq_ref[...] = jnp.sum(s * s, axis=0, keepdims=True)[None]


def _emb_stage1(x, w, b, block):
    n, k = x.shape
    nb = n // block
    rs = pl.BlockSpec((1, 1, HID), lambda i: (i, 0, 0))
    return pl.pallas_call(
        _emb_body,
        grid=(nb,),
        in_specs=[pl.BlockSpec((block, k), lambda i: (i, 0)),
                  pl.BlockSpec((k, HID), lambda i: (0, 0)),
                  pl.BlockSpec((1, HID), lambda i: (0, 0))],
        out_specs=[pl.BlockSpec((block, HID), lambda i: (i, 0)), rs, rs],
        out_shape=[jax.ShapeDtypeStruct((n, HID), F32),
                   jax.ShapeDtypeStruct((nb, 1, HID), F32),
                   jax.ShapeDtypeStruct((nb, 1, HID), F32)],
    )(x, w, b)


def _bnact_body(s_ref, mu_ref, iv_ref, g_ref, b_ref, o_ref):
    xn = (s_ref[...] - mu_ref[...]) * iv_ref[...] * g_ref[...] + b_ref[...]
    o_ref[...] = _silu(xn)


def _bnact(s, mu, iv, g, b, block):
    n = s.shape[0]
    bs = pl.BlockSpec((block, HID), lambda i: (i, 0))
    rs = pl.BlockSpec((1, HID), lambda i: (0, 0))
    return pl.pallas_call(
        _bnact_body,
        grid=(n // block,),
        in_specs=[bs, rs, rs, rs, rs],
        out_specs=bs,
        out_shape=jax.ShapeDtypeStruct((n, HID), F32),
    )(s, mu, iv, g, b)


def _smear_body(d_ref, o_ref, *, start, delta):
    off = start + delta * jax.lax.broadcasted_iota(
        jnp.int32, (1, HID), 1).astype(F32)
    diff = d_ref[...] - off
    o_ref[...] = jnp.exp((-0.5 / (delta * delta)) * diff * diff)


def _smearing(d, start, stop, block):
    n = d.shape[0]
    delta = (stop - start) / (HID - 1)
    return pl.pallas_call(
        functools.partial(_smear_body, start=start, delta=delta),
        grid=(n // block,),
        in_specs=[pl.BlockSpec((block, 1), lambda i: (i, 0))],
        out_specs=pl.BlockSpec((block, HID), lambda i: (i, 0)),
        out_shape=jax.ShapeDtypeStruct((n, HID), F32),
    )(d.reshape(n, 1))


def _heads_body(c_ref, w1_ref, b1_ref, w2_ref, b2_ref,
                w3_ref, b3_ref, w4_ref, b4_ref, o1_ref, o2_ref):
    c = c_ref[...]
    t1 = jax.lax.dot_general(c, w1_ref[...], (((1,), (0,)), ((), ())),
                             preferred_element_type=F32) + b1_ref[...]
    o1_ref[...] = jax.lax.dot_general(_silu(t1), w2_ref[...],
                                      (((1,), (0,)), ((), ())),
                                      preferred_element_type=F32) + b2_ref[...]
    t2 = jax.lax.dot_general(c, w3_ref[...], (((1,), (0,)), ((), ())),
                             preferred_element_type=F32) + b3_ref[...]
    o2_ref[...] = jax.lax.dot_general(_silu(t2), w4_ref[...],
                                      (((1,), (0,)), ((), ())),
                                      preferred_element_type=F32) + b4_ref[...]


def _heads(c, w1, b1, w2, b2, w3, b3, w4, b4):
    g = c.shape[0]
    full = lambda shape: pl.BlockSpec(shape, lambda: tuple(0 for _ in shape))
    return pl.pallas_call(
        _heads_body,
        in_specs=[full((g, HID)),
                  full((HID, HID)), full((1, HID)), full((HID, 1)), full((1, 1)),
                  full((HID, HID)), full((1, HID)), full((HID, 1)), full((1, 1))],
        out_specs=[full((g, 1)), full((g, 1))],
        out_shape=[jax.ShapeDtypeStruct((g, 1), F32),
                   jax.ShapeDtypeStruct((g, 1), F32)],
    )(c, w1, b1.reshape(1, HID), w2, b2.reshape(1, 1),
      w3, b3.reshape(1, HID), w4, b4.reshape(1, 1))


def _gated_layer(h, e, src, dst, p, eblock, nblock):
    n = h.shape[0]
    wabv = jnp.concatenate([p['A'], p['B'], p['V']], axis=1)
    hw = _matmul(h, wabv, nblock, out_dtype=BF16)
    hu = _matmul(h, p['U'], nblock)
    ga = jnp.take(hw[:, 0 * HID:1 * HID], src, axis=0)
    gb = jnp.take(hw[:, 1 * HID:2 * HID], dst, axis=0)
    gv = jnp.take(hw[:, 2 * HID:3 * HID], src, axis=0)
    sigma, sv = _edge_stage(ga, gb, gv, e, p['C'], eblock)
    num = jax.ops.segment_sum(sv, dst, num_segments=n,
                              indices_are_sorted=True)
    den = jax.ops.segment_sum(sigma, dst, num_segments=n,
                              indices_are_sorted=True)
    s, ps, pq = _node_stage1(hu, num, den, nblock)
    mu, iv = _finish_bn(ps, pq, n)
    return _node_stage2(h, s, mu, iv,
                        p['bn_g'].reshape(1, HID), p['bn_b'].reshape(1, HID),
                        nblock)


def kernel(x, edge_attr, angle_attr, edge_index, edge_index_lg, batch, params):
    n_nodes = x.shape[0]
    n_edges = edge_attr.shape[0]
    n_graphs = 256

    nblock = 2000
    eblock = 2000

    t, ps, pq = _emb_stage1(x, params['emb_w'],
                            params['emb_b'].reshape(1, HID), nblock)
    mu, iv = _finish_bn(ps, pq, n_nodes)
    h = _bnact(t, mu, iv, params['emb_bn_g'].reshape(1, HID),
               params['emb_bn_b'].reshape(1, HID), nblock)

    src, dst = edge_index[0], edge_index[1]
    perm = jnp.argsort(dst)
    srcp, dstp = jnp.take(src, perm), jnp.take(dst, perm)
    rank = jnp.zeros((n_edges,), jnp.int32).at[perm].set(
        jnp.arange(n_edges, dtype=jnp.int32), unique_indices=True,
        indices_are_sorted=False)
    src_lg2 = jnp.take(rank, edge_index_lg[0])
    dst_lg2 = jnp.take(rank, edge_index_lg[1])
    perm_lg = jnp.argsort(dst_lg2)
    src_lgp = jnp.take(src_lg2, perm_lg)
    dst_lgp = jnp.take(dst_lg2, perm_lg)

    m = _smearing(jnp.take(edge_attr, perm), 0.0, 6.0, eblock)
    a = _smearing(jnp.take(angle_attr, perm_lg), -1.0, 180.0, eblock)

    for i in range(len(params['atom'])):
        m = _gated_layer(m, a, src_lgp, dst_lgp, params['line'][i],
                         eblock, eblock)
        h = _gated_layer(h, m, srcp, dstp, params['atom'][i],
                         eblock, nblock)

    ones = jnp.ones((n_nodes,), F32)
    counts = jax.ops.segment_sum(ones, batch, num_segments=n_graphs)
    c = (jax.ops.segment_sum(h, batch, num_segments=n_graphs)
         / jnp.maximum(counts, 1.0)[:, None])

    out_bg, out_hull = _heads(
        c, params['bg_w1'], params['bg_b1'], params['bg_w2'], params['bg_b2'],
        params['hull_w1'], params['hull_b1'], params['hull_w2'],
        params['hull_b2'])
    return (out_bg, out_hull)
